# baseline (speedup 1.0000x reference)
import functools

import jax
import jax.numpy as jnp
from jax import lax
from jax.experimental import pallas as pl
from jax.experimental.pallas import tpu as pltpu
from jax.experimental.pallas import tpu_sc as plsc

NUM_BUCKETS = 1000000
EMBED_DIM = 16
BATCH = 16384

_NC = 2
_NS = 16
_NW = _NC * _NS
_LANES = 16

_BPW = BATCH // _NW
_NSTREAM = 4
_CHUNK = _BPW // _NSTREAM


def _body(x_hbm, table_hbm, wb_hbm, out_hbm, x_v, idx_v, rows_v, out_v,
          wb_v, sem):
    wid = lax.axis_index("s") * _NC + lax.axis_index("c")

    pltpu.sync_copy(x_hbm.at[wid], x_v)
    pltpu.sync_copy(wb_hbm, wb_v)

    for j in range(_NSTREAM):
        for k in range(_CHUNK // _LANES):
            v = x_v[j, pl.ds(k * _LANES, _LANES)]
            u = v.astype(jnp.uint32) * jnp.uint32(2654435761)
            h = u % jnp.uint32(NUM_BUCKETS)
            idx_v[j, pl.ds(k * _LANES, _LANES)] = h.astype(jnp.int32)

    copies = [
        pltpu.async_copy(
            table_hbm.at[idx_v.at[j]],
            rows_v.at[pl.ds(j * _CHUNK, _CHUNK)],
            sem,
        )
        for j in range(_NSTREAM)
    ]
    for c in copies:
        c.wait()

    w_vec = wb_v[0, :]
    b_vec = wb_v[1, :]
    iota = lax.iota(jnp.int32, _LANES)
    for g in range(_BPW // _LANES):
        rid = iota + (g * _LANES)
        acc = b_vec
        for d in range(EMBED_DIM):
            cid = jnp.full((_LANES,), d, jnp.int32)
            col = plsc.load_gather(rows_v, [rid, cid])
            acc = acc + col * w_vec[d]
        out_v[pl.ds(g * _LANES, _LANES)] = 1.0 / (1.0 + jnp.exp(-acc))

    pltpu.sync_copy(out_v, out_hbm.at[wid])


@jax.jit
def _run(x32, table, wb):
    mesh = plsc.VectorSubcoreMesh(core_axis_name="c", subcore_axis_name="s")
    f = pl.kernel(
        _body,
        out_type=jax.ShapeDtypeStruct((_NW, _BPW), jnp.float32),
        mesh=mesh,
        scratch_types=[
            pltpu.VMEM((_NSTREAM, _CHUNK), jnp.int32),
            pltpu.VMEM((_NSTREAM, _CHUNK), jnp.int32),
            pltpu.VMEM((_BPW, EMBED_DIM), jnp.float32),
            pltpu.VMEM((_BPW,), jnp.float32),
            pltpu.VMEM((2, EMBED_DIM), jnp.float32),
            pltpu.SemaphoreType.DMA,
        ],
        compiler_params=pltpu.CompilerParams(
            needs_layout_passes=False, use_tc_tiling_on_sc=False),
    )
    return f(x32, table, wb)


def kernel(x, table, W, b):
    x32 = x.astype(jnp.int32).reshape(_NW, _NSTREAM, _CHUNK)
    wb = jnp.concatenate(
        [W.reshape(1, EMBED_DIM),
         jnp.broadcast_to(b.reshape(1, 1), (1, EMBED_DIM))], axis=0)
    out = _run(x32, table, wb)
    return out.reshape(BATCH, 1)
